# baseline (speedup 1.0000x reference)
import functools

import jax
import jax.numpy as jnp
import numpy as np
from jax import lax
from jax.experimental import pallas as pl
from jax.experimental.pallas import tpu as pltpu
from jax.experimental.pallas import tpu_sc as plsc

NC = 2
NS = 16
NW = NC * NS
L = 16

CH, CW = 7, 7
NPB = CH * CW
BATCH = 16
PIX_PER_BATCH = BATCH * NPB
CHUNK = 112
NCHUNK = PIX_PER_BATCH // CHUNK
TPC = CHUNK // L


def _roi_kernel(nboxes, H, W, C):
    bpw = -(-nboxes // (NW * BATCH)) * BATCH
    nbatch = bpw // BATCH
    last_base = (nboxes - bpw) // 8 * 8
    hm1 = H - 1
    wm1 = W - 1

    mesh = plsc.VectorSubcoreMesh(core_axis_name="c", subcore_axis_name="s")

    def chunk_vmem():
        return [
            pltpu.VMEM((CHUNK,), jnp.int32),
            pltpu.VMEM((CHUNK,), jnp.int32),
            pltpu.VMEM((CHUNK,), jnp.int32),
            pltpu.VMEM((CHUNK,), jnp.int32),
            pltpu.VMEM((CHUNK,), jnp.float32),
            pltpu.VMEM((CHUNK,), jnp.float32),
            pltpu.VMEM((CHUNK,), jnp.float32),
            pltpu.VMEM((CHUNK,), jnp.float32),
            pltpu.VMEM((CHUNK, C), jnp.float32),
            pltpu.VMEM((CHUNK, C), jnp.float32),
            pltpu.VMEM((CHUNK, C), jnp.float32),
            pltpu.VMEM((CHUNK, C), jnp.float32),
            pltpu.VMEM((CHUNK, C), jnp.float32),
            pltpu.SemaphoreType.DMA,
            pltpu.SemaphoreType.DMA,
        ]

    @functools.partial(
        pl.kernel,
        out_type=jax.ShapeDtypeStruct((nboxes * NPB, C), jnp.float32),
        mesh=mesh,
        compiler_params=pltpu.CompilerParams(
            use_tc_tiling_on_sc=False, needs_layout_passes=False
        ),
        scratch_types=[
            pltpu.VMEM((bpw,), jnp.float32),
            pltpu.VMEM((bpw,), jnp.float32),
            pltpu.VMEM((bpw,), jnp.float32),
            pltpu.VMEM((bpw,), jnp.float32),
            pltpu.VMEM((CH * BATCH,), jnp.int32),
            pltpu.VMEM((CH * BATCH,), jnp.int32),
            pltpu.VMEM((CH * BATCH,), jnp.float32),
            pltpu.VMEM((CW * BATCH,), jnp.int32),
            pltpu.VMEM((CW * BATCH,), jnp.int32),
            pltpu.VMEM((CW * BATCH,), jnp.float32),
            pltpu.VMEM((PIX_PER_BATCH,), jnp.int32),
            pltpu.VMEM((PIX_PER_BATCH,), jnp.int32),
        ] + chunk_vmem() + chunk_vmem(),
    )
    def roi(cy1_h, cx1_h, cy2_h, cx2_h, ai_h, aj_h, img_h, out_h,
            cy1v, cx1v, cy2v, cx2v,
            y0r, y1r, wyv, x0v, x1v, wxv, aiv, ajv,
            *pp):
        bufs = (pp[:15], pp[15:])

        wid = lax.axis_index("s") * NC + lax.axis_index("c")
        base = jnp.minimum(wid * bpw, last_base)
        pltpu.sync_copy(cy1_h.at[pl.ds(base, bpw)], cy1v)
        pltpu.sync_copy(cx1_h.at[pl.ds(base, bpw)], cx1v)
        pltpu.sync_copy(cy2_h.at[pl.ds(base, bpw)], cy2v)
        pltpu.sync_copy(cx2_h.at[pl.ds(base, bpw)], cx2v)
        pltpu.sync_copy(ai_h, aiv)
        pltpu.sync_copy(aj_h, ajv)

        lanes = lax.iota(jnp.int32, L)

        def build_idx(q, dst):
            i00, i01, i10, i11, w00, w01, w10, w11 = dst[:8]

            def idx_body(t, _):
                ai = aiv[pl.ds(q * CHUNK + t * L, L)]
                aj = ajv[pl.ds(q * CHUNK + t * L, L)]
                ya = plsc.load_gather(y0r, [ai])
                yb = plsc.load_gather(y1r, [ai])
                xa = plsc.load_gather(x0v, [aj])
                xb = plsc.load_gather(x1v, [aj])
                i00[pl.ds(t * L, L)] = ya + xa
                i01[pl.ds(t * L, L)] = ya + xb
                i10[pl.ds(t * L, L)] = yb + xa
                i11[pl.ds(t * L, L)] = yb + xb
                wy = plsc.load_gather(wyv, [ai])
                wx = plsc.load_gather(wxv, [aj])
                wyx = wy * wx
                w00[pl.ds(t * L, L)] = 1.0 - wy - wx + wyx
                w01[pl.ds(t * L, L)] = wx - wyx
                w10[pl.ds(t * L, L)] = wy - wyx
                w11[pl.ds(t * L, L)] = wyx
                return 0

            lax.fori_loop(0, TPC, idx_body, 0)

        def fire_gathers(dst):
            i00, i01, i10, i11 = dst[:4]
            r00, r01, r10, r11 = dst[8:12]
            sem = dst[13]
            return [
                pltpu.async_copy(img_h.at[i00], r00, sem),
                pltpu.async_copy(img_h.at[i01], r01, sem),
                pltpu.async_copy(img_h.at[i10], r10, sem),
                pltpu.async_copy(img_h.at[i11], r11, sem),
            ]

        def blend(dst):
            w00, w01, w10, w11 = dst[4:8]
            r00, r01, r10, r11, outb = dst[8:13]

            def blend_t(t, _):
                lp = t * L + lanes
                v00 = w00[pl.ds(t * L, L)]
                v01 = w01[pl.ds(t * L, L)]
                v10 = w10[pl.ds(t * L, L)]
                v11 = w11[pl.ds(t * L, L)]

                def blend_c(c, cc):
                    acc = v00 * plsc.load_gather(r00, [lp, cc])
                    acc = acc + v01 * plsc.load_gather(r01, [lp, cc])
                    acc = acc + v10 * plsc.load_gather(r10, [lp, cc])
                    acc = acc + v11 * plsc.load_gather(r11, [lp, cc])
                    plsc.store_scatter(outb, [lp, cc], acc)
                    return cc + 1

                lax.fori_loop(0, C, blend_c, jnp.zeros((L,), jnp.int32),
                              unroll=8)
                return 0

            lax.fori_loop(0, TPC, blend_t, 0)

        def batch_body(b, _):
            bo = b * BATCH
            by1 = cy1v[pl.ds(bo, BATCH)]
            bx1 = cx1v[pl.ds(bo, BATCH)]
            by2 = cy2v[pl.ds(bo, BATCH)]
            bx2 = cx2v[pl.ds(bo, BATCH)]

            def grid_body(i, _):
                f = i.astype(jnp.float32) * (1.0 / (CH - 1))
                iny = by1 + f * (by2 - by1)
                yt = iny.astype(jnp.int32)
                wyv[pl.ds(i * L, L)] = iny - yt.astype(jnp.float32)
                y0r[pl.ds(i * L, L)] = jnp.minimum(yt, hm1) * W
                y1r[pl.ds(i * L, L)] = jnp.minimum(yt + 1, hm1) * W
                inx = bx1 + f * (bx2 - bx1)
                xt = inx.astype(jnp.int32)
                wxv[pl.ds(i * L, L)] = inx - xt.astype(jnp.float32)
                x0v[pl.ds(i * L, L)] = jnp.minimum(xt, wm1)
                x1v[pl.ds(i * L, L)] = jnp.minimum(xt + 1, wm1)
                return 0

            lax.fori_loop(0, CH, grid_body, 0)

            build_idx(0, bufs[0])
            pend_g = {0: fire_gathers(bufs[0])}
            pend_o = {}
            for q in range(NCHUNK):
                p = q % 2
                nxt = (q + 1) % 2
                if q + 1 < NCHUNK:
                    build_idx(q + 1, bufs[nxt])
                    pend_g[nxt] = fire_gathers(bufs[nxt])
                for cp in pend_g.pop(p):
                    cp.wait()
                if p in pend_o:
                    pend_o.pop(p).wait()
                blend(bufs[p])
                gp = (base + b * BATCH) * NPB + q * CHUNK
                pend_o[p] = pltpu.async_copy(
                    bufs[p][12], out_h.at[pl.ds(gp, CHUNK)], bufs[p][14]
                )
            for cp in pend_o.values():
                cp.wait()
            return 0

        lax.fori_loop(0, nbatch, batch_body, 0)

    return roi


@jax.jit
def kernel(metadata, image, boxes):
    B, H, W, C = image.shape
    n = boxes.shape[1]

    b = boxes[0].astype(jnp.float32)
    h = metadata[0, 0].astype(jnp.float32)
    w = metadata[0, 1].astype(jnp.float32)
    sy = (H - 1.0) / (h - 1.0)
    sx = (W - 1.0) / (w - 1.0)
    cx1 = b[:, 0] * sx
    cy1 = b[:, 1] * sy
    cx2 = b[:, 2] * sx
    cy2 = b[:, 3] * sy

    p = np.arange(PIX_PER_BATCH)
    nn = p // NPB
    k = p % NPB
    ai = jnp.asarray((k // CW) * L + nn, jnp.int32)
    aj = jnp.asarray((k % CW) * L + nn, jnp.int32)

    img_rows = image.reshape(H * W, C)
    out = _roi_kernel(n, H, W, C)(cy1, cx1, cy2, cx2, ai, aj, img_rows)
    return out.reshape(1, n, CH, CW, C)


# speedup vs baseline: 3.5032x; 3.5032x over previous
import functools

import jax
import jax.numpy as jnp
import numpy as np
from jax import lax
from jax.experimental import pallas as pl
from jax.experimental.pallas import tpu as pltpu
from jax.experimental.pallas import tpu_sc as plsc

NC = 2
NS = 16
NW = NC * NS
L = 16

CH, CW = 7, 7
NPB = CH * CW
BATCH = 16
PIX_PER_BATCH = BATCH * NPB
CHUNK = 112
NCHUNK = PIX_PER_BATCH // CHUNK
TPC = CHUNK // L


def _roi_kernel(nboxes, H, W, C):
    bpw = -(-nboxes // (NW * BATCH)) * BATCH
    nbatch = bpw // BATCH
    last_base = (nboxes - bpw) // 8 * 8
    hm1 = H - 1
    wm1 = W - 1

    mesh = plsc.VectorSubcoreMesh(core_axis_name="c", subcore_axis_name="s")

    def chunk_vmem():
        return [
            pltpu.VMEM((CHUNK,), jnp.int32),
            pltpu.VMEM((CHUNK,), jnp.int32),
            pltpu.VMEM((CHUNK,), jnp.int32),
            pltpu.VMEM((CHUNK,), jnp.int32),
            pltpu.VMEM((CHUNK,), jnp.float32),
            pltpu.VMEM((CHUNK,), jnp.float32),
            pltpu.VMEM((CHUNK,), jnp.float32),
            pltpu.VMEM((CHUNK,), jnp.float32),
            pltpu.VMEM((CHUNK, C), jnp.float32),
            pltpu.VMEM((CHUNK, C), jnp.float32),
            pltpu.VMEM((CHUNK, C), jnp.float32),
            pltpu.VMEM((CHUNK, C), jnp.float32),
            pltpu.VMEM((CHUNK, C), jnp.float32),
            pltpu.SemaphoreType.DMA,
            pltpu.SemaphoreType.DMA,
        ]

    @functools.partial(
        pl.kernel,
        out_type=jax.ShapeDtypeStruct((nboxes * NPB, C), jnp.float32),
        mesh=mesh,
        compiler_params=pltpu.CompilerParams(
            use_tc_tiling_on_sc=False, needs_layout_passes=False
        ),
        scratch_types=[
            pltpu.VMEM((bpw,), jnp.float32),
            pltpu.VMEM((bpw,), jnp.float32),
            pltpu.VMEM((bpw,), jnp.float32),
            pltpu.VMEM((bpw,), jnp.float32),
            pltpu.VMEM((CH * BATCH,), jnp.int32),
            pltpu.VMEM((CH * BATCH,), jnp.int32),
            pltpu.VMEM((CH * BATCH,), jnp.float32),
            pltpu.VMEM((CW * BATCH,), jnp.int32),
            pltpu.VMEM((CW * BATCH,), jnp.int32),
            pltpu.VMEM((CW * BATCH,), jnp.float32),
            pltpu.VMEM((PIX_PER_BATCH,), jnp.int32),
            pltpu.VMEM((PIX_PER_BATCH,), jnp.int32),
        ] + chunk_vmem() + chunk_vmem(),
    )
    def roi(cy1_h, cx1_h, cy2_h, cx2_h, ai_h, aj_h, img_h, out_h,
            cy1v, cx1v, cy2v, cx2v,
            y0r, y1r, wyv, x0v, x1v, wxv, aiv, ajv,
            *pp):
        bufs = (pp[:15], pp[15:])

        wid = lax.axis_index("s") * NC + lax.axis_index("c")
        base = jnp.minimum(wid * bpw, last_base)
        pltpu.sync_copy(cy1_h.at[pl.ds(base, bpw)], cy1v)
        pltpu.sync_copy(cx1_h.at[pl.ds(base, bpw)], cx1v)
        pltpu.sync_copy(cy2_h.at[pl.ds(base, bpw)], cy2v)
        pltpu.sync_copy(cx2_h.at[pl.ds(base, bpw)], cx2v)
        pltpu.sync_copy(ai_h, aiv)
        pltpu.sync_copy(aj_h, ajv)

        lanes = lax.iota(jnp.int32, L)

        def build_idx(q, dst):
            i00, i01, i10, i11, w00, w01, w10, w11 = dst[:8]

            def idx_body(t, _):
                ai = aiv[pl.ds(q * CHUNK + t * L, L)]
                aj = ajv[pl.ds(q * CHUNK + t * L, L)]
                ya = plsc.load_gather(y0r, [ai])
                yb = plsc.load_gather(y1r, [ai])
                xa = plsc.load_gather(x0v, [aj])
                xb = plsc.load_gather(x1v, [aj])
                i00[pl.ds(t * L, L)] = ya + xa
                i01[pl.ds(t * L, L)] = ya + xb
                i10[pl.ds(t * L, L)] = yb + xa
                i11[pl.ds(t * L, L)] = yb + xb
                wy = plsc.load_gather(wyv, [ai])
                wx = plsc.load_gather(wxv, [aj])
                wyx = wy * wx
                w00[pl.ds(t * L, L)] = 1.0 - wy - wx + wyx
                w01[pl.ds(t * L, L)] = wx - wyx
                w10[pl.ds(t * L, L)] = wy - wyx
                w11[pl.ds(t * L, L)] = wyx
                return 0

            lax.fori_loop(0, TPC, idx_body, 0)

        def fire_gathers(dst):
            i00, i01, i10, i11 = dst[:4]
            r00, r01, r10, r11 = dst[8:12]
            sem = dst[13]
            return [
                pltpu.async_copy(img_h.at[i00], r00, sem),
                pltpu.async_copy(img_h.at[i01], r01, sem),
                pltpu.async_copy(img_h.at[i10], r10, sem),
                pltpu.async_copy(img_h.at[i11], r11, sem),
            ]

        def blend(dst):
            w00, w01, w10, w11 = dst[4:8]
            r00, r01, r10, r11, outb = dst[8:13]

            def blend_t(t, _):
                lp = t * L + lanes
                v00 = w00[pl.ds(t * L, L)]
                v01 = w01[pl.ds(t * L, L)]
                v10 = w10[pl.ds(t * L, L)]
                v11 = w11[pl.ds(t * L, L)]

                def blend_c(c, cc):
                    acc = v00 * plsc.load_gather(r00, [lp, cc])
                    acc = acc + v01 * plsc.load_gather(r01, [lp, cc])
                    acc = acc + v10 * plsc.load_gather(r10, [lp, cc])
                    acc = acc + v11 * plsc.load_gather(r11, [lp, cc])
                    plsc.store_scatter(outb, [lp, cc], acc)
                    return cc + 1

                lax.fori_loop(0, C, blend_c, jnp.zeros((L,), jnp.int32),
                              unroll=8)
                return 0

            lax.fori_loop(0, TPC, blend_t, 0)

        def batch_body(b, _):
            bo = b * BATCH
            by1 = cy1v[pl.ds(bo, BATCH)]
            bx1 = cx1v[pl.ds(bo, BATCH)]
            by2 = cy2v[pl.ds(bo, BATCH)]
            bx2 = cx2v[pl.ds(bo, BATCH)]

            def grid_body(i, _):
                f = i.astype(jnp.float32) * (1.0 / (CH - 1))
                iny = by1 + f * (by2 - by1)
                yt = iny.astype(jnp.int32)
                wyv[pl.ds(i * L, L)] = iny - yt.astype(jnp.float32)
                y0r[pl.ds(i * L, L)] = jnp.minimum(yt, hm1) * W
                y1r[pl.ds(i * L, L)] = jnp.minimum(yt + 1, hm1) * W
                inx = bx1 + f * (bx2 - bx1)
                xt = inx.astype(jnp.int32)
                wxv[pl.ds(i * L, L)] = inx - xt.astype(jnp.float32)
                x0v[pl.ds(i * L, L)] = jnp.minimum(xt, wm1)
                x1v[pl.ds(i * L, L)] = jnp.minimum(xt + 1, wm1)
                return 0

            lax.fori_loop(0, CH, grid_body, 0)

            build_idx(0, bufs[0])
            pend_g = {0: fire_gathers(bufs[0])}
            pend_o = {}
            for q in range(NCHUNK):
                p = q % 2
                nxt = (q + 1) % 2
                if q + 1 < NCHUNK:
                    build_idx(q + 1, bufs[nxt])
                    pend_g[nxt] = fire_gathers(bufs[nxt])
                for cp in pend_g.pop(p):
                    cp.wait()
                if p in pend_o:
                    pend_o.pop(p).wait()
                gp = (base + b * BATCH) * NPB + q * CHUNK
                pend_o[p] = pltpu.async_copy(
                    bufs[p][8], out_h.at[pl.ds(gp, CHUNK)], bufs[p][14]
                )
            for cp in pend_o.values():
                cp.wait()
            return 0

        lax.fori_loop(0, nbatch, batch_body, 0)

    return roi


@jax.jit
def kernel(metadata, image, boxes):
    B, H, W, C = image.shape
    n = boxes.shape[1]

    b = boxes[0].astype(jnp.float32)
    h = metadata[0, 0].astype(jnp.float32)
    w = metadata[0, 1].astype(jnp.float32)
    sy = (H - 1.0) / (h - 1.0)
    sx = (W - 1.0) / (w - 1.0)
    cx1 = b[:, 0] * sx
    cy1 = b[:, 1] * sy
    cx2 = b[:, 2] * sx
    cy2 = b[:, 3] * sy

    p = np.arange(PIX_PER_BATCH)
    nn = p // NPB
    k = p % NPB
    ai = jnp.asarray((k // CW) * L + nn, jnp.int32)
    aj = jnp.asarray((k % CW) * L + nn, jnp.int32)

    img_rows = image.reshape(H * W, C)
    out = _roi_kernel(n, H, W, C)(cy1, cx1, cy2, cx2, ai, aj, img_rows)
    return out.reshape(1, n, CH, CW, C)
